# Initial kernel scaffold; baseline (speedup 1.0000x reference)
import jax
import jax.numpy as jnp
from jax.experimental import pallas as pl


def kernel(x, w1, b1, w2, b2, w3, b3):
    raise NotImplementedError("write your pallas kernel here")


# speedup vs baseline: 1.2847x; 1.2847x over previous
import functools

import jax
import jax.numpy as jnp
from jax.experimental import pallas as pl
from jax.experimental.pallas import tpu as pltpu

_LANE = 128
_SUBLANE = 8
_TILE_M = 2048


def _round_up(n, m):
    return (n + m - 1) // m * m


def _mlp_kernel(x_ref, w1_ref, b1_ref, w2_ref, b2_ref, w3_ref, b3_ref, o_ref):
    h = jnp.dot(x_ref[...], w1_ref[...], preferred_element_type=jnp.float32)
    h = jnp.maximum(h + b1_ref[...], 0.0)
    h = jnp.dot(h, w2_ref[...], preferred_element_type=jnp.float32)
    h = jnp.maximum(h + b2_ref[...], 0.0)
    h = jnp.dot(h, w3_ref[...], preferred_element_type=jnp.float32)
    o_ref[...] = jnp.tanh(h + b3_ref[...]).astype(o_ref.dtype)


@functools.partial(jax.jit, static_argnames=("tile_m",))
def _forward(x, w1, b1, w2, b2, w3, b3, *, tile_m=_TILE_M):
    B, num_inputs = x.shape
    num_actions = w3.shape[1]
    x = x.astype(jnp.float32)

    rows = _round_up(max(B, 1), _SUBLANE)
    tile_m = min(tile_m, _round_up(rows, _SUBLANE))
    B_pad = _round_up(rows, tile_m)
    n_steps = B_pad // tile_m

    x_pad = jnp.pad(x, ((0, B_pad - B), (0, 0)))
    n_out_pad = _round_up(num_actions, _LANE)
    w3p = jnp.pad(w3.astype(jnp.float32), ((0, 0), (0, n_out_pad - num_actions)))
    b3p = jnp.pad(b3.astype(jnp.float32), ((0, 0), (0, n_out_pad - num_actions)))

    resident = lambda a: pl.BlockSpec(a.shape, lambda i: (0, 0))
    in_specs = [
        pl.BlockSpec((tile_m, num_inputs), lambda i: (i, 0)),
        resident(w1), resident(b1),
        resident(w2), resident(b2),
        resident(w3p), resident(b3p),
    ]
    out_spec = pl.BlockSpec((tile_m, n_out_pad), lambda i: (i, 0))

    semantics = ("arbitrary",)

    out_pad = pl.pallas_call(
        _mlp_kernel,
        out_shape=jax.ShapeDtypeStruct((B_pad, n_out_pad), jnp.float32),
        grid=(n_steps,),
        in_specs=in_specs,
        out_specs=out_spec,
        compiler_params=pltpu.CompilerParams(
            dimension_semantics=semantics,
            vmem_limit_bytes=64 * 1024 * 1024,
        ),
    )(x_pad,
      w1.astype(jnp.float32), b1.astype(jnp.float32),
      w2.astype(jnp.float32), b2.astype(jnp.float32),
      w3p, b3p)

    return out_pad[:B, :num_actions]


def kernel(x, w1, b1, w2, b2, w3, b3):
    return _forward(x, w1, b1, w2, b2, w3, b3)


# speedup vs baseline: 1.3572x; 1.0565x over previous
import functools

import jax
import jax.numpy as jnp
from jax.experimental import pallas as pl
from jax.experimental.pallas import tpu as pltpu

_LANE = 128
_SUBLANE = 8
_TILE_M = 4096


def _round_up(n, m):
    return (n + m - 1) // m * m


def _mlp_kernel(x_ref, w1_ref, b1_ref, w2_ref, b2_ref, w3_ref, b3_ref, o_ref):
    h = jnp.dot(x_ref[...], w1_ref[...], preferred_element_type=jnp.float32)
    h = jnp.maximum(h + b1_ref[...], 0.0)
    h = jnp.dot(h, w2_ref[...], preferred_element_type=jnp.float32)
    h = jnp.maximum(h + b2_ref[...], 0.0)
    h = jnp.dot(h, w3_ref[...], preferred_element_type=jnp.float32)
    o_ref[...] = jnp.tanh(h + b3_ref[...]).astype(o_ref.dtype)


@functools.partial(jax.jit, static_argnames=("tile_m",))
def _forward(x, w1, b1, w2, b2, w3, b3, *, tile_m=_TILE_M):
    B, num_inputs = x.shape
    num_actions = w3.shape[1]
    x = x.astype(jnp.float32)

    rows = _round_up(max(B, 1), _SUBLANE)
    tile_m = min(tile_m, _round_up(rows, _SUBLANE))
    B_pad = _round_up(rows, tile_m)
    n_steps = B_pad // tile_m

    x_pad = jnp.pad(x, ((0, B_pad - B), (0, 0)))
    n_out_pad = _round_up(num_actions, _LANE)
    w3p = jnp.pad(w3.astype(jnp.float32), ((0, 0), (0, n_out_pad - num_actions)))
    b3p = jnp.pad(b3.astype(jnp.float32), ((0, 0), (0, n_out_pad - num_actions)))

    resident = lambda a: pl.BlockSpec(a.shape, lambda i: (0, 0))
    in_specs = [
        pl.BlockSpec((tile_m, num_inputs), lambda i: (i, 0)),
        resident(w1), resident(b1),
        resident(w2), resident(b2),
        resident(w3p), resident(b3p),
    ]
    out_spec = pl.BlockSpec((tile_m, n_out_pad), lambda i: (i, 0))

    semantics = ("arbitrary",)

    out_pad = pl.pallas_call(
        _mlp_kernel,
        out_shape=jax.ShapeDtypeStruct((B_pad, n_out_pad), jnp.float32),
        grid=(n_steps,),
        in_specs=in_specs,
        out_specs=out_spec,
        compiler_params=pltpu.CompilerParams(
            dimension_semantics=semantics,
            vmem_limit_bytes=64 * 1024 * 1024,
        ),
    )(x_pad,
      w1.astype(jnp.float32), b1.astype(jnp.float32),
      w2.astype(jnp.float32), b2.astype(jnp.float32),
      w3p, b3p)

    return out_pad[:B, :num_actions]


def kernel(x, w1, b1, w2, b2, w3, b3):
    return _forward(x, w1, b1, w2, b2, w3, b3)


# speedup vs baseline: 1.3634x; 1.0046x over previous
import functools

import jax
import jax.numpy as jnp
from jax.experimental import pallas as pl
from jax.experimental.pallas import tpu as pltpu

_LANE = 128
_SUBLANE = 8
_TILE_M = 8192


def _round_up(n, m):
    return (n + m - 1) // m * m


def _mlp_kernel(x_ref, w1_ref, b1_ref, w2_ref, b2_ref, w3_ref, b3_ref, o_ref):
    h = jnp.dot(x_ref[...], w1_ref[...], preferred_element_type=jnp.float32)
    h = jnp.maximum(h + b1_ref[...], 0.0)
    h = jnp.dot(h, w2_ref[...], preferred_element_type=jnp.float32)
    h = jnp.maximum(h + b2_ref[...], 0.0)
    h = jnp.dot(h, w3_ref[...], preferred_element_type=jnp.float32)
    o_ref[...] = jnp.tanh(h + b3_ref[...]).astype(o_ref.dtype)


@functools.partial(jax.jit, static_argnames=("tile_m",))
def _forward(x, w1, b1, w2, b2, w3, b3, *, tile_m=_TILE_M):
    B, num_inputs = x.shape
    num_actions = w3.shape[1]
    x = x.astype(jnp.float32)

    rows = _round_up(max(B, 1), _SUBLANE)
    tile_m = min(tile_m, _round_up(rows, _SUBLANE))
    B_pad = _round_up(rows, tile_m)
    n_steps = B_pad // tile_m

    x_pad = jnp.pad(x, ((0, B_pad - B), (0, 0)))
    n_out_pad = _round_up(num_actions, _LANE)
    w3p = jnp.pad(w3.astype(jnp.float32), ((0, 0), (0, n_out_pad - num_actions)))
    b3p = jnp.pad(b3.astype(jnp.float32), ((0, 0), (0, n_out_pad - num_actions)))

    resident = lambda a: pl.BlockSpec(a.shape, lambda i: (0, 0))
    in_specs = [
        pl.BlockSpec((tile_m, num_inputs), lambda i: (i, 0)),
        resident(w1), resident(b1),
        resident(w2), resident(b2),
        resident(w3p), resident(b3p),
    ]
    out_spec = pl.BlockSpec((tile_m, n_out_pad), lambda i: (i, 0))

    semantics = ("arbitrary",)

    out_pad = pl.pallas_call(
        _mlp_kernel,
        out_shape=jax.ShapeDtypeStruct((B_pad, n_out_pad), jnp.float32),
        grid=(n_steps,),
        in_specs=in_specs,
        out_specs=out_spec,
        compiler_params=pltpu.CompilerParams(
            dimension_semantics=semantics,
            vmem_limit_bytes=64 * 1024 * 1024,
        ),
    )(x_pad,
      w1.astype(jnp.float32), b1.astype(jnp.float32),
      w2.astype(jnp.float32), b2.astype(jnp.float32),
      w3p, b3p)

    return out_pad[:B, :num_actions]


def kernel(x, w1, b1, w2, b2, w3, b3):
    return _forward(x, w1, b1, w2, b2, w3, b3)
